# baseline (speedup 1.0000x reference)
import jax
import jax.numpy as jnp
from jax import lax
from jax.experimental import pallas as pl
from jax.experimental.pallas import tpu as pltpu
from jax.experimental.pallas import tpu_sc as plsc

N_SPK = 100000
D = 2048
B = 16384
L = 16

NC, NS = 2, 16
NW = NC * NS
B_PER_W = B // NW
CHUNK = 16
N_CHUNKS = B_PER_W // CHUNK
N_SLICES = D // L

_MAGIC = 0x5F3759DF


def _sc_body(sid_hbm, table_hbm, out_hbm, idx_v, buf0, buf1, sem0, sem1):
    wid = lax.axis_index("s") * NC + lax.axis_index("c")
    base = wid * B_PER_W
    pltpu.sync_copy(sid_hbm.at[pl.ds(base, B_PER_W)], idx_v)

    def chunk_idx(cc):
        return idx_v[pl.ds(cc * CHUNK, CHUNK)]

    def gather_start(cc, buf, sem):
        pltpu.async_copy(table_hbm.at[chunk_idx(cc)], buf, sem)

    def gather_wait(cc, buf, sem):
        pltpu.make_async_copy(table_hbm.at[chunk_idx(cc)], buf, sem).wait()

    def normalize_chunk(buf):
        def row_body(r, _):
            def acc_body(j, acc):
                x = buf[r, pl.ds(j * L, L)]
                return acc + x * x

            acc = lax.fori_loop(
                0, N_SLICES, acc_body, jnp.zeros((L,), jnp.float32)
            )
            lane = lax.iota(jnp.int32, L)
            s = acc
            for sft in (1, 2, 4, 8):
                s = s + s.at[(lane + sft) & (L - 1)].get(
                    mode="promise_in_bounds")
            magic = jnp.full((L,), _MAGIC, jnp.int32)
            s_bits = lax.bitcast_convert_type(s, jnp.int32)
            y = lax.bitcast_convert_type(magic - (s_bits >> 1), jnp.float32)
            half_s = 0.5 * s
            for _unused in range(3):
                y = y * (1.5 - half_s * y * y)
            r_inv = jnp.minimum(y, jnp.float32(1e12))

            def scale_body(j, _2):
                x = buf[r, pl.ds(j * L, L)]
                buf[r, pl.ds(j * L, L)] = x * r_inv
                return 0

            lax.fori_loop(1, N_SLICES, scale_body, 0)
            x0 = buf[r, pl.ds(0, L)] * r_inv
            lane = lax.iota(jnp.int32, L)
            buf[r, pl.ds(0, L)] = jnp.where(lane == 0, jnp.float32(1.0), x0)
            return 0

        lax.fori_loop(0, CHUNK, row_body, 0)

    gather_start(0, buf0, sem0)

    def step(c2, _):
        c = c2 * 2
        for k in range(2):
            cc = c + k
            buf, sem = (buf0, sem0) if k == 0 else (buf1, sem1)
            nbuf, nsem = (buf1, sem1) if k == 0 else (buf0, sem0)

            @pl.when(cc + 1 < N_CHUNKS)
            def _prefetch():
                gather_start(cc + 1, nbuf, nsem)

            gather_wait(cc, buf, sem)
            normalize_chunk(buf)
            pltpu.sync_copy(buf, out_hbm.at[pl.ds(base + cc * CHUNK, CHUNK)])
        return 0

    lax.fori_loop(0, N_CHUNKS // 2, step, 0)


@jax.jit
def _reverb_filter_bank(sid, table):
    mesh = plsc.VectorSubcoreMesh(core_axis_name="c", subcore_axis_name="s")
    return pl.kernel(
        _sc_body,
        out_type=jax.ShapeDtypeStruct((B, D), jnp.float32),
        mesh=mesh,
        scratch_types=[
            pltpu.VMEM((B_PER_W,), jnp.int32),
            pltpu.VMEM((CHUNK, D), jnp.float32),
            pltpu.VMEM((CHUNK, D), jnp.float32),
            pltpu.SemaphoreType.DMA,
            pltpu.SemaphoreType.DMA,
        ],
    )(sid, table)


def kernel(sid, table):
    return _reverb_filter_bank(sid.astype(jnp.int32), table)


# speedup vs baseline: 2.1964x; 2.1964x over previous
import jax
import jax.numpy as jnp
from jax import lax
from jax.experimental import pallas as pl
from jax.experimental.pallas import tpu as pltpu
from jax.experimental.pallas import tpu_sc as plsc

N_SPK = 100000
D = 2048
B = 16384
L = 16

NC, NS = 2, 16
NW = NC * NS
B_PER_W = B // NW
CHUNK = 16
N_CHUNKS = B_PER_W // CHUNK
N_SLICES = D // L
U = 8

_MAGIC = 0x5F3759DF


def _sc_body(sid_hbm, table_hbm, out_hbm, idx_v, buf0, buf1,
             gsem0, gsem1, ssem0, ssem1):
    wid = lax.axis_index("s") * NC + lax.axis_index("c")
    base = wid * B_PER_W
    pltpu.sync_copy(sid_hbm.at[pl.ds(base, B_PER_W)], idx_v)

    def chunk_idx(cc):
        return idx_v[pl.ds(cc * CHUNK, CHUNK)]

    def gather_start(cc, buf, sem):
        pltpu.async_copy(table_hbm.at[chunk_idx(cc)], buf, sem)

    def gather_wait(cc, buf, sem):
        pltpu.make_async_copy(table_hbm.at[chunk_idx(cc)], buf, sem).wait()

    def store_start(cc, buf, sem):
        pltpu.make_async_copy(
            buf, out_hbm.at[pl.ds(base + cc * CHUNK, CHUNK)], sem).start()

    def store_wait(cc, buf, sem):
        pltpu.make_async_copy(
            buf, out_hbm.at[pl.ds(base + cc * CHUNK, CHUNK)], sem).wait()

    def normalize_chunk(buf):
        lane = lax.iota(jnp.int32, L)
        magic = jnp.full((L,), _MAGIC, jnp.int32)
        one = jnp.full((L,), 1.0, jnp.float32)

        def row_body(r, _):
            def acc_body(j2, accs):
                j = j2 * U
                return tuple(
                    a + buf[r, pl.ds((j + u) * L, L)] *
                    buf[r, pl.ds((j + u) * L, L)]
                    for u, a in enumerate(accs)
                )

            zeros = tuple(jnp.zeros((L,), jnp.float32) for _ in range(U))
            accs = lax.fori_loop(0, N_SLICES // U, acc_body, zeros)
            a0 = (accs[0] + accs[1]) + (accs[2] + accs[3])
            a1 = (accs[4] + accs[5]) + (accs[6] + accs[7])
            s = a0 + a1
            for sft in (1, 2, 4, 8):
                s = s + s.at[(lane + sft) & (L - 1)].get(
                    mode="promise_in_bounds")
            s_bits = lax.bitcast_convert_type(s, jnp.int32)
            y = lax.bitcast_convert_type(magic - (s_bits >> 1), jnp.float32)
            half_s = 0.5 * s
            for _unused in range(3):
                y = y * (1.5 - half_s * y * y)
            r_inv = jnp.minimum(y, jnp.float32(1e12))

            def scale_body(j2, _2):
                j = j2 * U
                for u in range(U):
                    sl = pl.ds((j + u) * L, L)
                    buf[r, sl] = buf[r, sl] * r_inv
                return 0

            lax.fori_loop(0, N_SLICES // U, scale_body, 0)
            x0 = buf[r, pl.ds(0, L)]
            buf[r, pl.ds(0, L)] = jnp.where(lane == 0, one, x0)
            return 0

        lax.fori_loop(0, CHUNK, row_body, 0)

    gather_start(0, buf0, gsem0)

    def step(c2, _):
        c = c2 * 2
        for k in range(2):
            cc = c + k
            buf, gsem, ssem = ((buf0, gsem0, ssem0) if k == 0
                               else (buf1, gsem1, ssem1))
            nbuf, ngsem, nssem = ((buf1, gsem1, ssem1) if k == 0
                                  else (buf0, gsem0, ssem0))

            @pl.when(cc + 1 < N_CHUNKS)
            def _prefetch():
                @pl.when(cc >= 1)
                def _drain():
                    store_wait(cc - 1, nbuf, nssem)

                gather_start(cc + 1, nbuf, ngsem)

            gather_wait(cc, buf, gsem)
            normalize_chunk(buf)
            store_start(cc, buf, ssem)
        return 0

    lax.fori_loop(0, N_CHUNKS // 2, step, 0)
    store_wait(N_CHUNKS - 2, buf0, ssem0)
    store_wait(N_CHUNKS - 1, buf1, ssem1)


@jax.jit
def _reverb_filter_bank(sid, table):
    mesh = plsc.VectorSubcoreMesh(core_axis_name="c", subcore_axis_name="s")
    return pl.kernel(
        _sc_body,
        out_type=jax.ShapeDtypeStruct((B, D), jnp.float32),
        mesh=mesh,
        scratch_types=[
            pltpu.VMEM((B_PER_W,), jnp.int32),
            pltpu.VMEM((CHUNK, D), jnp.float32),
            pltpu.VMEM((CHUNK, D), jnp.float32),
            pltpu.SemaphoreType.DMA,
            pltpu.SemaphoreType.DMA,
            pltpu.SemaphoreType.DMA,
            pltpu.SemaphoreType.DMA,
        ],
    )(sid, table)


def kernel(sid, table):
    return _reverb_filter_bank(sid.astype(jnp.int32), table)


# speedup vs baseline: 4.2799x; 1.9486x over previous
import jax
import jax.numpy as jnp
from jax import lax
from jax.experimental import pallas as pl
from jax.experimental.pallas import tpu as pltpu
from jax.experimental.pallas import tpu_sc as plsc

N_SPK = 100000
D = 2048
B = 16384
L = 16

NC, NS = 2, 16
NW = NC * NS
B_PER_W = B // NW
CHUNK = 16
N_CHUNKS = B_PER_W // CHUNK
N_SLICES = D // L
U = 8

_MAGIC = 0x5F3759DF


def _sc_body(sid_hbm, table_hbm, out_hbm, idx_v, buf0, buf1,
             gsem0, gsem1, ssem0, ssem1):
    wid = lax.axis_index("s") * NC + lax.axis_index("c")
    base = wid * B_PER_W
    pltpu.sync_copy(sid_hbm.at[pl.ds(base, B_PER_W)], idx_v)

    def chunk_idx(cc):
        return idx_v[pl.ds(cc * CHUNK, CHUNK)]

    def gather_start(cc, buf, sem):
        pltpu.async_copy(table_hbm.at[chunk_idx(cc)], buf, sem)

    def gather_wait(cc, buf, sem):
        pltpu.make_async_copy(table_hbm.at[chunk_idx(cc)], buf, sem).wait()

    def store_start(cc, buf, sem):
        pltpu.make_async_copy(
            buf, out_hbm.at[pl.ds(base + cc * CHUNK, CHUNK)], sem).start()

    def store_wait(cc, buf, sem):
        pltpu.make_async_copy(
            buf, out_hbm.at[pl.ds(base + cc * CHUNK, CHUNK)], sem).wait()

    def normalize_chunk(buf):
        lane = lax.iota(jnp.int32, L)
        magic = jnp.full((L,), _MAGIC, jnp.int32)
        one = jnp.full((L,), 1.0, jnp.float32)

        svec = jnp.zeros((L,), jnp.float32)
        for r in range(CHUNK):
            def acc_body(j2, accs, r=r):
                j = j2 * U
                return tuple(
                    a + buf[r, pl.ds((j + u) * L, L)] *
                    buf[r, pl.ds((j + u) * L, L)]
                    for u, a in enumerate(accs)
                )

            zeros = tuple(jnp.zeros((L,), jnp.float32) for _ in range(U))
            accs = lax.fori_loop(0, N_SLICES // U, acc_body, zeros)
            a0 = (accs[0] + accs[1]) + (accs[2] + accs[3])
            a1 = (accs[4] + accs[5]) + (accs[6] + accs[7])
            s = a0 + a1
            for sft in (1, 2, 4, 8):
                s = s + s.at[(lane + sft) & (L - 1)].get(
                    mode="promise_in_bounds")
            svec = jnp.where(lane == r, s, svec)

        s_bits = lax.bitcast_convert_type(svec, jnp.int32)
        y = lax.bitcast_convert_type(magic - (s_bits >> 1), jnp.float32)
        half_s = 0.5 * svec
        for _unused in range(3):
            y = y * (1.5 - half_s * y * y)
        r_inv_vec = jnp.minimum(y, jnp.float32(1e12))

        for r in range(CHUNK):
            rv = r_inv_vec.at[jnp.full((L,), r, jnp.int32)].get(
                mode="promise_in_bounds")

            def scale_body(j2, _2, r=r, rv=rv):
                j = j2 * U
                for u in range(U):
                    sl = pl.ds((j + u) * L, L)
                    buf[r, sl] = buf[r, sl] * rv
                return 0

            lax.fori_loop(0, N_SLICES // U, scale_body, 0)
            x0 = buf[r, pl.ds(0, L)]
            buf[r, pl.ds(0, L)] = jnp.where(lane == 0, one, x0)

    gather_start(0, buf0, gsem0)

    def step(c2, _):
        c = c2 * 2
        for k in range(2):
            cc = c + k
            buf, gsem, ssem = ((buf0, gsem0, ssem0) if k == 0
                               else (buf1, gsem1, ssem1))
            nbuf, ngsem, nssem = ((buf1, gsem1, ssem1) if k == 0
                                  else (buf0, gsem0, ssem0))

            @pl.when(cc + 1 < N_CHUNKS)
            def _prefetch():
                @pl.when(cc >= 1)
                def _drain():
                    store_wait(cc - 1, nbuf, nssem)

                gather_start(cc + 1, nbuf, ngsem)

            gather_wait(cc, buf, gsem)
            normalize_chunk(buf)
            store_start(cc, buf, ssem)
        return 0

    lax.fori_loop(0, N_CHUNKS // 2, step, 0)
    store_wait(N_CHUNKS - 2, buf0, ssem0)
    store_wait(N_CHUNKS - 1, buf1, ssem1)


@jax.jit
def _reverb_filter_bank(sid, table):
    mesh = plsc.VectorSubcoreMesh(core_axis_name="c", subcore_axis_name="s")
    return pl.kernel(
        _sc_body,
        out_type=jax.ShapeDtypeStruct((B, D), jnp.float32),
        mesh=mesh,
        scratch_types=[
            pltpu.VMEM((B_PER_W,), jnp.int32),
            pltpu.VMEM((CHUNK, D), jnp.float32),
            pltpu.VMEM((CHUNK, D), jnp.float32),
            pltpu.SemaphoreType.DMA,
            pltpu.SemaphoreType.DMA,
            pltpu.SemaphoreType.DMA,
            pltpu.SemaphoreType.DMA,
        ],
    )(sid, table)


def kernel(sid, table):
    return _reverb_filter_bank(sid.astype(jnp.int32), table)
